# baseline (speedup 1.0000x reference)
import functools

import jax
import jax.numpy as jnp
from jax import lax
from jax.experimental import pallas as pl
from jax.experimental.pallas import tpu as pltpu
from jax.experimental.pallas import tpu_sc as plsc

N_NODES = 10000
N_EDGES = 320000
D_IN = 128
D_HID = 16
D_OUT = 5

NC = 2
NS = 16
NW = NC * NS
CH = 128
NBUF = 8
NINF = NBUF // 2
K = NBUF * (-(-N_EDGES // (NW * CH * NBUF)))
E_PAD = NW * K * CH
N_ACC = 10240
ZR = N_ACC // NS
VEC = 16

_mesh = plsc.VectorSubcoreMesh(core_axis_name="c", subcore_axis_name="s")
_sc_params = pltpu.CompilerParams(use_tc_tiling_on_sc=False,
                                  needs_layout_passes=False)


def _rsqrt_nr(d):
  half = d * 0.5
  i = plsc.bitcast(d, jnp.int32)
  i = jnp.full((VEC,), 0x5F3759DF, jnp.int32) - lax.shift_right_logical(i, 1)
  y = plsc.bitcast(i, jnp.float32)
  for _ in range(3):
    y = y * (1.5 - half * y * y)
  return y


def _zero_accum(s, zbuf, accum):
  def zrow(i, _):
    zbuf[i, :] = jnp.zeros((D_HID,), jnp.float32)
    return ()
  lax.fori_loop(0, ZR, zrow, ())
  pltpu.sync_copy(zbuf, accum.at[pl.ds(s * ZR, ZR)])


def _deg_pass(dst_v, ones_v, accum, ssem):
  def body(jj, _):
    for r in range(NBUF):
      j = jj * NBUF + r

      @pl.when(jj > 0)
      def _():
        pltpu.make_async_copy(ones_v, accum.at[dst_v.at[j - NBUF]],
                              ssem[r]).wait()

      pltpu.async_copy(ones_v, accum.at[dst_v.at[j]], ssem[r], add=True)
    return ()

  lax.fori_loop(0, K // NBUF, body, ())
  for i in range(K - NBUF, K):
    pltpu.make_async_copy(ones_v, accum.at[dst_v.at[i]],
                          ssem[i % NBUF]).wait()


def _agg_pass(table2d, src_v, dst_v, rows, accum, gsem, ssem):
  for r in range(NINF):
    pltpu.async_copy(table2d.at[src_v.at[r]], rows[r], gsem[r])

  def body(jj, _):
    for r in range(NBUF):
      j = jj * NBUF + r
      rd = (r + NINF) % NBUF
      pltpu.make_async_copy(table2d.at[src_v.at[j]], rows[r], gsem[r]).wait()
      pltpu.async_copy(rows[r], accum.at[dst_v.at[j]], ssem[r], add=True)

      @pl.when(j >= NINF)
      def _():
        pltpu.make_async_copy(rows[rd], accum.at[dst_v.at[j - NINF]],
                              ssem[rd]).wait()

      @pl.when(j + NINF < K)
      def _():
        pltpu.async_copy(table2d.at[src_v.at[j + NINF]], rows[rd], gsem[rd])
    return ()

  lax.fori_loop(0, K // NBUF, body, ())
  for i in range(K - NINF, K):
    pltpu.make_async_copy(rows[i % NBUF], accum.at[dst_v.at[i]],
                          ssem[i % NBUF]).wait()


def _flush_accum(c, s, accum, out_hbm):
  pltpu.sync_copy(accum.at[pl.ds(s * ZR, ZR)],
                  out_hbm.at[c, pl.ds(s * ZR, ZR)])


_scratch_common = [
    pltpu.VMEM((K, CH), jnp.int32),
    pltpu.VMEM((K, CH), jnp.int32),
    [pltpu.VMEM((CH, D_HID), jnp.float32)] * NBUF,
    pltpu.VMEM((ZR, D_HID), jnp.float32),
    pltpu.VMEM((ZR, D_HID), jnp.float32),
    pltpu.VMEM((ZR, D_HID), jnp.float32),
    pltpu.VMEM_SHARED((N_ACC, D_HID), jnp.float32),
    [pltpu.SemaphoreType.DMA] * NBUF,
    [pltpu.SemaphoreType.DMA] * NBUF,
]


@functools.partial(
    pl.kernel,
    out_type=(
        jax.ShapeDtypeStruct((NC, N_ACC, D_HID), jnp.float32),
        jax.ShapeDtypeStruct((NC, N_ACC, D_HID), jnp.float32),
        jax.ShapeDtypeStruct((NC, N_ACC, D_HID), jnp.float32),
    ),
    mesh=_mesh,
    compiler_params=_sc_params,
    scratch_types=_scratch_common,
)
def _sc_mega1(xw1_hbm, src_hbm, dst_hbm, agg_hbm, p_hbm, dinv_hbm,
              src_v, dst_v, rows, zbuf, buf_a, buf_b, accum, gsem, ssem):
  c = lax.axis_index("c")
  s = lax.axis_index("s")
  wid = s * NC + c
  _zero_accum(s, zbuf, accum)

  def orow(i, _):
    rows[0][i, :] = jnp.ones((D_HID,), jnp.float32)
    return ()
  lax.fori_loop(0, CH, orow, ())
  plsc.subcore_barrier()

  pltpu.sync_copy(dst_hbm.at[s * NC], dst_v)
  _deg_pass(dst_v, rows[0], accum, ssem)
  pltpu.sync_copy(dst_hbm.at[s * NC + 1], dst_v)
  _deg_pass(dst_v, rows[0], accum, ssem)
  plsc.subcore_barrier()
  pltpu.sync_copy(src_hbm.at[wid], src_v)
  pltpu.sync_copy(dst_hbm.at[wid], dst_v)

  pltpu.sync_copy(accum.at[pl.ds(s * ZR, ZR)], buf_a)
  pltpu.sync_copy(xw1_hbm.at[pl.ds(s * ZR, ZR)], buf_b)

  def prow(i, _):
    dinv = _rsqrt_nr(buf_a[i, :] + 1.0)
    buf_a[i, :] = dinv
    buf_b[i, :] = buf_b[i, :] * dinv
    return ()
  lax.fori_loop(0, ZR, prow, ())
  pltpu.sync_copy(buf_a, dinv_hbm.at[c, pl.ds(s * ZR, ZR)])
  pltpu.sync_copy(buf_b, p_hbm.at[c, pl.ds(s * ZR, ZR)])
  _zero_accum(s, zbuf, accum)
  plsc.subcore_barrier()

  _agg_pass(p_hbm.at[c], src_v, dst_v, rows, accum, gsem, ssem)
  plsc.subcore_barrier()
  _flush_accum(c, s, accum, agg_hbm)


@functools.partial(
    pl.kernel,
    out_type=(
        jax.ShapeDtypeStruct((NC, N_ACC, D_HID), jnp.float32),
        jax.ShapeDtypeStruct((NC, N_ACC, D_HID), jnp.float32),
    ),
    mesh=_mesh,
    compiler_params=_sc_params,
    scratch_types=_scratch_common + [pltpu.VMEM((VEC,), jnp.float32)],
)
def _sc_mega2(agg1_hbm, p_hbm, dinv_hbm, b1_hbm, src_hbm, dst_hbm,
              agg_hbm, q_hbm,
              src_v, dst_v, rows, zbuf, buf_a, buf_b, accum, gsem, ssem,
              b1_v):
  c = lax.axis_index("c")
  s = lax.axis_index("s")
  wid = s * NC + c
  _zero_accum(s, zbuf, accum)
  pltpu.sync_copy(src_hbm.at[wid], src_v)
  pltpu.sync_copy(dst_hbm.at[wid], dst_v)
  pltpu.sync_copy(b1_hbm, b1_v)

  pltpu.sync_copy(agg1_hbm.at[0, pl.ds(s * ZR, ZR)], buf_a)
  pltpu.sync_copy(p_hbm.at[c, pl.ds(s * ZR, ZR)], buf_b)
  b1 = b1_v[...]

  def qrow0(i, _):
    buf_b[i, :] = buf_b[i, :] + buf_a[i, :]
    return ()
  lax.fori_loop(0, ZR, qrow0, ())
  pltpu.sync_copy(agg1_hbm.at[1, pl.ds(s * ZR, ZR)], buf_a)

  def qrow1(i, _):
    buf_b[i, :] = buf_b[i, :] + buf_a[i, :]
    return ()
  lax.fori_loop(0, ZR, qrow1, ())
  pltpu.sync_copy(dinv_hbm.at[c, pl.ds(s * ZR, ZR)], buf_a)

  def qrow2(i, _):
    dinv = buf_a[i, :]
    buf_b[i, :] = jnp.maximum(dinv * buf_b[i, :] + b1, 0.0) * dinv
    return ()
  lax.fori_loop(0, ZR, qrow2, ())
  pltpu.sync_copy(buf_b, q_hbm.at[c, pl.ds(s * ZR, ZR)])
  plsc.subcore_barrier()

  _agg_pass(q_hbm.at[c], src_v, dst_v, rows, accum, gsem, ssem)
  plsc.subcore_barrier()
  _flush_accum(c, s, accum, agg_hbm)


def _tca_body(x_ref, w1_ref, out_ref):
  out_ref[0:N_NODES, :] = jnp.dot(x_ref[...], w1_ref[...],
                                  preferred_element_type=jnp.float32)
  out_ref[N_NODES:, :] = jnp.zeros((N_ACC - N_NODES, D_HID), jnp.float32)


def _tcz_body(agg_ref, q_ref, dinv_ref, w2_ref, b2_ref, out_ref):
  su = (agg_ref[0, :N_NODES, :] + agg_ref[1, :N_NODES, :]
        + q_ref[0, :N_NODES, :])
  z = jnp.dot(dinv_ref[0, :N_NODES, :] * su, w2_ref[...],
              preferred_element_type=jnp.float32) + b2_ref[...]
  m = jnp.max(z, axis=1, keepdims=True)
  zs = z - m
  out_ref[...] = zs - jnp.log(jnp.sum(jnp.exp(zs), axis=1, keepdims=True))


def kernel(x, edge_index, W1, b1, W2, b2):
  src = jnp.concatenate(
      [edge_index[0], jnp.zeros((E_PAD - N_EDGES,), jnp.int32)])
  dst = jnp.concatenate(
      [edge_index[1], jnp.full((E_PAD - N_EDGES,), N_NODES, jnp.int32)])
  src3 = src.reshape(NW, K, CH)
  dst3 = dst.reshape(NW, K, CH)

  xw1 = pl.pallas_call(
      _tca_body,
      out_shape=jax.ShapeDtypeStruct((N_ACC, D_HID), jnp.float32),
  )(x, W1)

  agg1, p, dinv = _sc_mega1(xw1, src3, dst3)
  agg2, q = _sc_mega2(agg1, p, dinv, b1, src3, dst3)

  out = pl.pallas_call(
      _tcz_body,
      out_shape=jax.ShapeDtypeStruct((N_NODES, D_OUT), jnp.float32),
  )(agg2, q, dinv, W2, b2.reshape(1, D_OUT))
  return out


# speedup vs baseline: 1.0696x; 1.0696x over previous
import functools

import jax
import jax.numpy as jnp
from jax import lax
from jax.experimental import pallas as pl
from jax.experimental.pallas import tpu as pltpu
from jax.experimental.pallas import tpu_sc as plsc

N_NODES = 10000
N_EDGES = 320000
D_IN = 128
D_HID = 16
D_OUT = 5

NC = 2
NS = 16
NW = NC * NS
CH = 128
NBUF = 8
NINF = NBUF // 2
K = NBUF * (-(-N_EDGES // (NW * CH * NBUF)))
E_PAD = NW * K * CH
N_ACC = 10240
ZR = N_ACC // NS
VEC = 16
UNR = 8

_mesh = plsc.VectorSubcoreMesh(core_axis_name="c", subcore_axis_name="s")
_sc_params = pltpu.CompilerParams(use_tc_tiling_on_sc=False,
                                  needs_layout_passes=False)


def _rsqrt_nr(d):
  half = d * 0.5
  i = plsc.bitcast(d, jnp.int32)
  i = jnp.full((VEC,), 0x5F3759DF, jnp.int32) - lax.shift_right_logical(i, 1)
  y = plsc.bitcast(i, jnp.float32)
  for _ in range(3):
    y = y * (1.5 - half * y * y)
  return y


def _deg_pass(dst_v, ones_v, accum, ssem):
  def body(jj, _):
    for r in range(NBUF):
      j = jj * NBUF + r

      @pl.when(jj > 0)
      def _():
        pltpu.make_async_copy(ones_v, accum.at[dst_v.at[j - NBUF]],
                              ssem[r]).wait()

      pltpu.async_copy(ones_v, accum.at[dst_v.at[j]], ssem[r], add=True)
    return ()

  lax.fori_loop(0, K // NBUF, body, ())
  for i in range(K - NBUF, K):
    pltpu.make_async_copy(ones_v, accum.at[dst_v.at[i]],
                          ssem[i % NBUF]).wait()


def _agg_pass(table2d, src_v, dst_v, rows, accum, gsem, ssem):
  for r in range(NINF):
    pltpu.async_copy(table2d.at[src_v.at[r]], rows[r], gsem[r])

  def body(jj, _):
    for r in range(NBUF):
      j = jj * NBUF + r
      rd = (r + NINF) % NBUF
      pltpu.make_async_copy(table2d.at[src_v.at[j]], rows[r], gsem[r]).wait()
      pltpu.async_copy(rows[r], accum.at[dst_v.at[j]], ssem[r], add=True)

      @pl.when(j >= NINF)
      def _():
        pltpu.make_async_copy(rows[rd], accum.at[dst_v.at[j - NINF]],
                              ssem[rd]).wait()

      @pl.when(j + NINF < K)
      def _():
        pltpu.async_copy(table2d.at[src_v.at[j + NINF]], rows[rd], gsem[rd])
    return ()

  lax.fori_loop(0, K // NBUF, body, ())
  for i in range(K - NINF, K):
    pltpu.make_async_copy(rows[i % NBUF], accum.at[dst_v.at[i]],
                          ssem[i % NBUF]).wait()


def _flush_accum(c, s, accum, out_hbm):
  plsc.subcore_barrier()
  pltpu.sync_copy(accum.at[pl.ds(s * ZR, ZR)],
                  out_hbm.at[c, pl.ds(s * ZR, ZR)])


@functools.partial(
    pl.kernel,
    out_type=jax.ShapeDtypeStruct((NC, N_ACC, D_HID), jnp.float32),
    mesh=_mesh,
    compiler_params=_sc_params,
    scratch_types=[
        pltpu.VMEM((K, CH), jnp.int32),
        pltpu.VMEM((CH, D_HID), jnp.float32),
        pltpu.VMEM_SHARED((N_ACC, D_HID), jnp.float32),
        [pltpu.SemaphoreType.DMA] * NBUF,
    ],
)
def _sc_degree(dst_hbm, ones_hbm, zeros_hbm, out_hbm, dst_v, ones_v, accum,
               ssem):
  c = lax.axis_index("c")
  s = lax.axis_index("s")
  wid = s * NC + c
  pltpu.sync_copy(zeros_hbm, accum.at[pl.ds(s * ZR, ZR)])
  pltpu.sync_copy(ones_hbm, ones_v)
  pltpu.sync_copy(dst_hbm.at[wid], dst_v)
  plsc.subcore_barrier()
  _deg_pass(dst_v, ones_v, accum, ssem)
  _flush_accum(c, s, accum, out_hbm)


_scratch_mega = [
    pltpu.VMEM((K, CH), jnp.int32),
    pltpu.VMEM((K, CH), jnp.int32),
    [pltpu.VMEM((CH, D_HID), jnp.float32)] * NBUF,
    pltpu.VMEM((ZR, D_HID), jnp.float32),
    pltpu.VMEM((ZR, D_HID), jnp.float32),
    pltpu.VMEM((ZR, D_HID), jnp.float32),
    pltpu.VMEM((ZR, D_HID), jnp.float32),
    pltpu.VMEM_SHARED((N_ACC, D_HID), jnp.float32),
    [pltpu.SemaphoreType.DMA] * NBUF,
    [pltpu.SemaphoreType.DMA] * NBUF,
]


@functools.partial(
    pl.kernel,
    out_type=(
        jax.ShapeDtypeStruct((NC, N_ACC, D_HID), jnp.float32),
        jax.ShapeDtypeStruct((NC, N_ACC, D_HID), jnp.float32),
        jax.ShapeDtypeStruct((NC, N_ACC, D_HID), jnp.float32),
    ),
    mesh=_mesh,
    compiler_params=_sc_params,
    scratch_types=_scratch_mega,
)
def _sc_mega1(xw1_hbm, deg_hbm, src_hbm, dst_hbm, zeros_hbm,
              agg_hbm, p_hbm, dinv_hbm,
              src_v, dst_v, rows, buf_a, buf_b, buf_c, buf_d,
              accum, gsem, ssem):
  c = lax.axis_index("c")
  s = lax.axis_index("s")
  wid = s * NC + c
  sl = pl.ds(s * ZR, ZR)
  pltpu.sync_copy(zeros_hbm, accum.at[sl])
  pltpu.sync_copy(src_hbm.at[wid], src_v)
  pltpu.sync_copy(dst_hbm.at[wid], dst_v)

  pltpu.sync_copy(deg_hbm.at[0, sl], buf_a)
  pltpu.sync_copy(deg_hbm.at[1, sl], buf_b)
  pltpu.sync_copy(xw1_hbm.at[sl], buf_c)

  def prow(ii, _):
    for u in range(UNR):
      i = ii * UNR + u
      dinv = _rsqrt_nr(buf_a[i, :] + buf_b[i, :] + 1.0)
      buf_a[i, :] = dinv
      buf_c[i, :] = buf_c[i, :] * dinv
    return ()
  lax.fori_loop(0, ZR // UNR, prow, ())
  pltpu.sync_copy(buf_a, dinv_hbm.at[c, sl])
  pltpu.sync_copy(buf_c, p_hbm.at[c, sl])
  plsc.subcore_barrier()

  _agg_pass(p_hbm.at[c], src_v, dst_v, rows, accum, gsem, ssem)
  _flush_accum(c, s, accum, agg_hbm)


@functools.partial(
    pl.kernel,
    out_type=(
        jax.ShapeDtypeStruct((NC, N_ACC, D_HID), jnp.float32),
        jax.ShapeDtypeStruct((NC, N_ACC, D_HID), jnp.float32),
    ),
    mesh=_mesh,
    compiler_params=_sc_params,
    scratch_types=_scratch_mega + [pltpu.VMEM((VEC,), jnp.float32)],
)
def _sc_mega2(agg1_hbm, p_hbm, dinv_hbm, b1_hbm, src_hbm, dst_hbm, zeros_hbm,
              agg_hbm, q_hbm,
              src_v, dst_v, rows, buf_a, buf_b, buf_c, buf_d,
              accum, gsem, ssem, b1_v):
  c = lax.axis_index("c")
  s = lax.axis_index("s")
  wid = s * NC + c
  sl = pl.ds(s * ZR, ZR)
  pltpu.sync_copy(zeros_hbm, accum.at[sl])
  pltpu.sync_copy(src_hbm.at[wid], src_v)
  pltpu.sync_copy(dst_hbm.at[wid], dst_v)
  pltpu.sync_copy(b1_hbm, b1_v)

  pltpu.sync_copy(agg1_hbm.at[0, sl], buf_a)
  pltpu.sync_copy(agg1_hbm.at[1, sl], buf_b)
  pltpu.sync_copy(p_hbm.at[c, sl], buf_c)
  pltpu.sync_copy(dinv_hbm.at[c, sl], buf_d)
  b1 = b1_v[...]

  def qrow(ii, _):
    for u in range(UNR):
      i = ii * UNR + u
      dinv = buf_d[i, :]
      su = buf_a[i, :] + buf_b[i, :] + buf_c[i, :]
      buf_c[i, :] = jnp.maximum(dinv * su + b1, 0.0) * dinv
    return ()
  lax.fori_loop(0, ZR // UNR, qrow, ())
  pltpu.sync_copy(buf_c, q_hbm.at[c, sl])
  plsc.subcore_barrier()

  _agg_pass(q_hbm.at[c], src_v, dst_v, rows, accum, gsem, ssem)
  _flush_accum(c, s, accum, agg_hbm)


def _tca_body(x_ref, w1_ref, out_ref):
  out_ref[0:N_NODES, :] = jnp.dot(x_ref[...], w1_ref[...],
                                  preferred_element_type=jnp.float32)
  out_ref[N_NODES:, :] = jnp.zeros((N_ACC - N_NODES, D_HID), jnp.float32)


def _tcz_body(agg_ref, q_ref, dinv_ref, w2_ref, b2_ref, out_ref):
  su = (agg_ref[0, :N_NODES, :] + agg_ref[1, :N_NODES, :]
        + q_ref[0, :N_NODES, :])
  z = jnp.dot(dinv_ref[0, :N_NODES, :] * su, w2_ref[...],
              preferred_element_type=jnp.float32) + b2_ref[...]
  m = jnp.max(z, axis=1, keepdims=True)
  zs = z - m
  out_ref[...] = zs - jnp.log(jnp.sum(jnp.exp(zs), axis=1, keepdims=True))


def kernel(x, edge_index, W1, b1, W2, b2):
  src = jnp.concatenate(
      [edge_index[0], jnp.zeros((E_PAD - N_EDGES,), jnp.int32)])
  dst = jnp.concatenate(
      [edge_index[1], jnp.full((E_PAD - N_EDGES,), N_NODES, jnp.int32)])
  src3 = src.reshape(NW, K, CH)
  dst3 = dst.reshape(NW, K, CH)
  zeros_rows = jnp.zeros((ZR, D_HID), jnp.float32)
  ones_rows = jnp.ones((CH, D_HID), jnp.float32)

  deg16 = _sc_degree(dst3, ones_rows, zeros_rows)

  xw1 = pl.pallas_call(
      _tca_body,
      out_shape=jax.ShapeDtypeStruct((N_ACC, D_HID), jnp.float32),
  )(x, W1)

  agg1, p, dinv = _sc_mega1(xw1, deg16, src3, dst3, zeros_rows)
  agg2, q = _sc_mega2(agg1, p, dinv, b1, src3, dst3, zeros_rows)

  out = pl.pallas_call(
      _tcz_body,
      out_shape=jax.ShapeDtypeStruct((N_NODES, D_OUT), jnp.float32),
  )(agg2, q, dinv, W2, b2.reshape(1, D_OUT))
  return out
